# baseline (speedup 1.0000x reference)
import functools

import jax
import jax.numpy as jnp
from jax import lax
from jax.experimental import pallas as pl
from jax.experimental.pallas import tpu as pltpu
from jax.experimental.pallas import tpu_sc as plsc

B, C, X, Y = 4, 96, 256, 256
N = 131072
BXY = B * X * Y
NC, NS, L = 2, 16, 16
NW = NC * NS
PTS_PER_W = N // NW
DST_PER_W = BXY // NW
ROWS_PER_W = DST_PER_W // Y
PCHUNK = 8192
SCH = 512
CCH = 1024


def _scatter_body(coords_hbm, feats_hbm, nhwc_hbm, p_hbm, cbuf, rowbuf,
                  pbuf, sem, ssem):
    w = lax.axis_index("s") * NC + lax.axis_index("c")
    base = w * PTS_PER_W
    iota = lax.iota(jnp.int32, L)
    zero16 = jnp.zeros((L,), jnp.int32)

    def stage(ring, ch):
        pltpu.async_copy(
            feats_hbm.at[pl.ds(base + ch * SCH, SCH), :],
            rowbuf.at[ring], ssem)

    def stage_wait(ring, ch):
        pltpu.make_async_copy(
            feats_hbm.at[pl.ds(base + ch * SCH, SCH), :],
            rowbuf.at[ring], ssem).wait()

    stage(0, 0)
    pltpu.sync_copy(coords_hbm.at[:, pl.ds(base, PTS_PER_W)], cbuf)

    @plsc.parallel_loop(0, PTS_PER_W // L, unroll=4)
    def _(j):
        bb = cbuf[0, pl.ds(j * L, L)]
        xx = cbuf[1, pl.ds(j * L, L)]
        yy = cbuf[2, pl.ds(j * L, L)]
        pbuf[pl.ds(j * L, L)] = (bb * X + xx) * Y + yy

    pltpu.sync_copy(pbuf, p_hbm.at[pl.ds(base, PTS_PER_W)])

    def scatter_chunk(ring, ch):
        stage_wait(ring, ch)
        cb = ch * SCH

        def g_body(g, _):
            pv = pbuf[pl.ds(cb + g * L, L)]
            for l in range(L):
                pj = jnp.sum(jnp.where(iota == l, pv, 0))
                pltpu.async_copy(rowbuf.at[ring, g * L + l],
                                 nhwc_hbm.at[pj], sem)
            return 0

        lax.fori_loop(0, SCH // L, g_body, 0)

        def d_body(j, _):
            pltpu.make_async_copy(rowbuf.at[ring, 0],
                                  nhwc_hbm.at[0], sem).wait()
            return 0

        lax.fori_loop(0, SCH, d_body, 0)

    for ch in range(PTS_PER_W // SCH):
        if ch + 1 < PTS_PER_W // SCH:
            stage((ch + 1) % 2, ch + 1)
        scatter_chunk(ch % 2, ch)


def _transpose_body(p_hbm, nhwc_hbm, out_hbm, idxbuf, pbuf, rows, outb,
                    sem, osem):
    w = lax.axis_index("s") * NC + lax.axis_index("c")
    dbase = w * DST_PER_W
    b = w // (X // ROWS_PER_W)
    x0 = (w % (X // ROWS_PER_W)) * ROWS_PER_W
    iota = lax.iota(jnp.int32, L)
    zero16 = jnp.zeros((L,), jnp.int32)
    zf16 = jnp.zeros((L,), jnp.float32)

    with jax.named_scope("clear"):
        @plsc.parallel_loop(0, DST_PER_W // L, unroll=8)
        def _(g):
            idxbuf[pl.ds(g * L, L)] = zero16

    with jax.named_scope("scan"):
        for chunk in range(N // PCHUNK):
            pltpu.sync_copy(p_hbm.at[pl.ds(chunk * PCHUNK, PCHUNK)], pbuf)

            @plsc.parallel_loop(0, PCHUNK // L, unroll=4)
            def _(j):
                v = pbuf[pl.ds(j * L, L)]
                rel = v - dbase
                m = (rel >= 0) & (rel < DST_PER_W)
                relc = jnp.clip(rel, 0, DST_PER_W - 1)
                plsc.store_scatter(idxbuf, [relc], iota + 1, mask=m)

    def stage(ring, sb):
        pltpu.async_copy(
            nhwc_hbm.at[pl.ds(dbase + sb * Y, Y), :], rows.at[ring], sem)

    def stage_wait(ring, sb):
        pltpu.make_async_copy(
            nhwc_hbm.at[pl.ds(dbase + sb * Y, Y), :], rows.at[ring],
            sem).wait()

    def owrite(ring, sb, start):
        x = x0 + sb
        for half in range(2):
            dst = out_hbm.at[b, :, x >> 3, half, x & 7, :]
            src = outb.at[ring, :, pl.ds(half * 128, 128)]
            if start:
                pltpu.async_copy(src, dst, osem)
            else:
                pltpu.make_async_copy(src, dst, osem).wait()

    def flush(ring, sb):
        with jax.named_scope("gwait"):
            stage_wait(ring, sb)

        @pl.when(sb >= 2)
        def _():
            owrite(ring, sb - 2, False)

        rbase = sb * Y
        with jax.named_scope("transpose"):
            for cg in range(C // L):
                c0 = cg * L

                @plsc.parallel_loop(0, Y // L, unroll=2)
                def _(g):
                    posv = g * L + iota
                    ibv = idxbuf[pl.ds(rbase + g * L, L)]
                    m = ibv > 0
                    for d in range(L):
                        ch = (iota + d) & (L - 1)
                        vals = plsc.load_gather(rows.at[ring],
                                                [posv, c0 + ch])
                        plsc.store_scatter(outb.at[ring], [c0 + ch, posv],
                                           jnp.where(m, vals, zf16))
        with jax.named_scope("writeout"):
            owrite(ring, sb, True)

    with jax.named_scope("prime"):
        stage(0, 0)

    def pair_body(t, _):
        sb0 = 2 * t
        stage(1, sb0 + 1)
        flush(0, sb0)

        @pl.when(t < ROWS_PER_W // 2 - 1)
        def _():
            stage(0, sb0 + 2)

        flush(1, sb0 + 1)
        return 0

    lax.fori_loop(0, ROWS_PER_W // 2, pair_body, 0)
    owrite(0, ROWS_PER_W - 2, False)
    owrite(1, ROWS_PER_W - 1, False)


@functools.cache
def _build():
    mesh = plsc.VectorSubcoreMesh(core_axis_name="c", subcore_axis_name="s")
    cparams = pltpu.CompilerParams(needs_layout_passes=False,
                                   use_tc_tiling_on_sc=False)
    k2 = pl.kernel(
        _scatter_body,
        out_type=(
            jax.ShapeDtypeStruct((BXY, C), jnp.float32),
            jax.ShapeDtypeStruct((N,), jnp.int32),
        ),
        mesh=mesh,
        compiler_params=cparams,
        scratch_types=[
            pltpu.VMEM((3, PTS_PER_W), jnp.int32),
            pltpu.VMEM((2, SCH, C), jnp.float32),
            pltpu.VMEM((PTS_PER_W,), jnp.int32),
            pltpu.SemaphoreType.DMA,
            pltpu.SemaphoreType.DMA,
        ],
    )
    k3 = pl.kernel(
        _transpose_body,
        out_type=jax.ShapeDtypeStruct((B, C, X // 8, Y // 128, 8, 128),
                                      jnp.float32),
        mesh=mesh,
        compiler_params=cparams,
        scratch_types=[
            pltpu.VMEM((DST_PER_W,), jnp.int32),
            pltpu.VMEM((PCHUNK,), jnp.int32),
            pltpu.VMEM((2, Y, C), jnp.float32),
            pltpu.VMEM((2, C, Y), jnp.float32),
            pltpu.SemaphoreType.DMA,
            pltpu.SemaphoreType.DMA,
        ],
    )
    return k2, k3


def kernel(feats, coords):
    k2, k3 = _build()
    nhwc, p = k2(coords.astype(jnp.int32).T, feats)
    out6 = k3(p, nhwc)
    return out6.transpose(0, 1, 2, 4, 3, 5).reshape(B, C, X, Y)
